# baseline (speedup 1.0000x reference)
import functools

import jax
import jax.numpy as jnp
from jax import lax
from jax.experimental import pallas as pl
from jax.experimental.pallas import tpu as pltpu
from jax.experimental.pallas import tpu_sc as plsc

VOCAB = 100000
EMBED = 64
HID = 16
OUT = 2
BATCH = 16384
HIST = 200
CHUNK_A = 104
CHUNK_B = 96
PACK = 128 // HID


def _mm_body(t_ref, w_ref, o_ref):
    o_ref[:] = jnp.dot(t_ref[0], w_ref[:], preferred_element_type=jnp.float32)[None]


def _fold_table(table, W1):
    grid = 50
    pk_rows = VOCAB // PACK // grid
    t8 = table.reshape(grid, pk_rows, PACK * EMBED)
    w1big = jnp.kron(jnp.eye(PACK, dtype=W1.dtype), W1)
    packed = pl.pallas_call(
        _mm_body,
        grid=(grid,),
        in_specs=[
            pl.BlockSpec((1, pk_rows, PACK * EMBED), lambda i: (i, 0, 0)),
            pl.BlockSpec((PACK * EMBED, PACK * HID), lambda i: (0, 0)),
        ],
        out_specs=pl.BlockSpec((1, pk_rows, PACK * HID), lambda i: (i, 0, 0)),
        out_shape=jax.ShapeDtypeStruct((grid, pk_rows, PACK * HID), jnp.float32),
    )(t8, w1big)
    return packed.reshape(VOCAB, HID)


NBUF = 4


def _make_sc_pool():
    info = plsc.get_sparse_core_info()
    nc, ns = info.num_cores, info.num_subcores
    nw = nc * ns
    bpw = BATCH // nw
    mesh = plsc.VectorSubcoreMesh(core_axis_name="c", subcore_axis_name="s")

    @functools.partial(
        pl.kernel,
        out_type=jax.ShapeDtypeStruct((BATCH, HID), jnp.float32),
        mesh=mesh,
        scratch_types=[
            pltpu.VMEM((bpw * HIST,), jnp.int32),
            pltpu.VMEM((NBUF, HIST, HID), jnp.float32),
            pltpu.VMEM((bpw, HID), jnp.float32),
            [pltpu.SemaphoreType.DMA] * NBUF,
        ],
        compiler_params=pltpu.CompilerParams(use_tc_tiling_on_sc=False),
    )
    def sc_pool(x_hbm, t1_hbm, out_hbm, idx_v, bufs, out_v, sems):
        wid = lax.axis_index("s") * nc + lax.axis_index("c")
        base = wid * bpw
        pltpu.sync_copy(x_hbm.at[pl.ds(base * HIST, bpw * HIST)], idx_v)

        def issue(slot, r):
            pltpu.async_copy(
                t1_hbm.at[idx_v.at[pl.ds(r * HIST, CHUNK_A)]],
                bufs.at[slot, pl.ds(0, CHUNK_A)],
                sems[slot],
            )
            pltpu.async_copy(
                t1_hbm.at[idx_v.at[pl.ds(r * HIST + CHUNK_A, CHUNK_B)]],
                bufs.at[slot, pl.ds(CHUNK_A, CHUNK_B)],
                sems[slot],
            )

        def drain(slot):
            pltpu.make_async_copy(
                t1_hbm.at[pl.ds(0, HIST)],
                bufs.at[slot],
                sems[slot],
            ).wait()

        for b in range(NBUF):
            issue(b, b)

        def accumulate(slot, out_row):
            accs = [jnp.zeros((HID,), jnp.float32)] * 8
            for j in range(HIST):
                accs[j % 8] = accs[j % 8] + bufs[slot, j]
            out_v[out_row] = (
                ((accs[0] + accs[1]) + (accs[2] + accs[3]))
                + ((accs[4] + accs[5]) + (accs[6] + accs[7]))
            )

        def steady(r0, _):
            for b in range(NBUF):
                r = r0 + b
                drain(b)
                accumulate(b, r)
                issue(b, r + NBUF)
            return 0

        lax.fori_loop(0, (bpw - NBUF) // NBUF, lambda i, c: steady(i * NBUF, c), 0)
        for b in range(NBUF):
            drain(b)
            accumulate(b, bpw - NBUF + b)
        pltpu.sync_copy(out_v, out_hbm.at[pl.ds(base, bpw)])

    return sc_pool


def _head_body(s_ref, b1_ref, w2_ref, b2_ref, o_ref):
    h = jnp.maximum(s_ref[:] * (1.0 / HIST) + b1_ref[:], 0.0)
    o_ref[:] = jnp.dot(h, w2_ref[:], preferred_element_type=jnp.float32) + b2_ref[:]


def _head(sums, b1, W2, b2):
    rows_blk = 2048
    grid = BATCH // rows_blk
    return pl.pallas_call(
        _head_body,
        grid=(grid,),
        in_specs=[
            pl.BlockSpec((rows_blk, HID), lambda i: (i, 0)),
            pl.BlockSpec((1, HID), lambda i: (0, 0)),
            pl.BlockSpec((HID, OUT), lambda i: (0, 0)),
            pl.BlockSpec((1, OUT), lambda i: (0, 0)),
        ],
        out_specs=pl.BlockSpec((rows_blk, OUT), lambda i: (i, 0)),
        out_shape=jax.ShapeDtypeStruct((BATCH, OUT), jnp.float32),
    )(sums, b1.reshape(1, HID), W2, b2.reshape(1, OUT))


def kernel(x, table, W1, b1, W2, b2):
    table1 = _fold_table(table, W1)
    sums = _make_sc_pool()(x.astype(jnp.int32).reshape(-1), table1)
    return _head(sums, b1, W2, b2)


# speedup vs baseline: 1.0164x; 1.0164x over previous
import functools

import jax
import jax.numpy as jnp
from jax import lax
from jax.experimental import pallas as pl
from jax.experimental.pallas import tpu as pltpu
from jax.experimental.pallas import tpu_sc as plsc

VOCAB = 100000
EMBED = 64
HID = 16
OUT = 2
BATCH = 16384
HIST = 200
CHUNK_A = 104
CHUNK_B = 96
PACK = 128 // HID


def _mm_body(t_ref, w_ref, o_ref):
    o_ref[:] = jnp.dot(t_ref[0], w_ref[:], preferred_element_type=jnp.float32)[None]


def _fold_table(table, W1):
    grid = 50
    pk_rows = VOCAB // PACK // grid
    t8 = table.reshape(grid, pk_rows, PACK * EMBED)
    w1big = jnp.kron(jnp.eye(PACK, dtype=W1.dtype), W1)
    packed = pl.pallas_call(
        _mm_body,
        grid=(grid,),
        in_specs=[
            pl.BlockSpec((1, pk_rows, PACK * EMBED), lambda i: (i, 0, 0)),
            pl.BlockSpec((PACK * EMBED, PACK * HID), lambda i: (0, 0)),
        ],
        out_specs=pl.BlockSpec((1, pk_rows, PACK * HID), lambda i: (i, 0, 0)),
        out_shape=jax.ShapeDtypeStruct((grid, pk_rows, PACK * HID), jnp.float32),
    )(t8, w1big)
    return packed.reshape(VOCAB, HID)


NBUF = 4


def _make_sc_pool():
    info = plsc.get_sparse_core_info()
    nc, ns = info.num_cores, info.num_subcores
    nw = nc * ns
    bpw = BATCH // nw
    mesh = plsc.VectorSubcoreMesh(core_axis_name="c", subcore_axis_name="s")

    @functools.partial(
        pl.kernel,
        out_type=jax.ShapeDtypeStruct((BATCH, HID), jnp.float32),
        mesh=mesh,
        scratch_types=[
            pltpu.VMEM((bpw * HIST,), jnp.int32),
            pltpu.VMEM((NBUF, HIST, HID), jnp.float32),
            pltpu.VMEM((bpw, HID), jnp.float32),
            [pltpu.SemaphoreType.DMA] * NBUF,
        ],
        compiler_params=pltpu.CompilerParams(use_tc_tiling_on_sc=False),
    )
    def sc_pool(x_hbm, t1_hbm, out_hbm, idx_v, bufs, out_v, sems):
        wid = lax.axis_index("s") * nc + lax.axis_index("c")
        base = wid * bpw
        pltpu.sync_copy(x_hbm.at[pl.ds(base * HIST, bpw * HIST)], idx_v)

        def issue(slot, r):
            pltpu.async_copy(
                t1_hbm.at[idx_v.at[pl.ds(r * HIST, CHUNK_A)]],
                bufs.at[slot, pl.ds(0, CHUNK_A)],
                sems[slot],
            )
            pltpu.async_copy(
                t1_hbm.at[idx_v.at[pl.ds(r * HIST + CHUNK_A, CHUNK_B)]],
                bufs.at[slot, pl.ds(CHUNK_A, CHUNK_B)],
                sems[slot],
            )

        def drain(slot):
            pltpu.make_async_copy(
                t1_hbm.at[pl.ds(0, HIST)],
                bufs.at[slot],
                sems[slot],
            ).wait()

        for b in range(NBUF):
            issue(b, b)

        def accumulate(slot, out_row):
            accs = [jnp.zeros((HID,), jnp.float32)] * 8
            for j in range(HIST):
                accs[j % 8] = accs[j % 8] + bufs[slot, j]
            out_v[out_row] = (
                ((accs[0] + accs[1]) + (accs[2] + accs[3]))
                + ((accs[4] + accs[5]) + (accs[6] + accs[7]))
            )

        def steady(r0, _):
            for b in range(NBUF):
                r = r0 + b
                drain(b)
                accumulate(b, r)
                issue(b, r + NBUF)
            return 0

        lax.fori_loop(0, (bpw - NBUF) // NBUF, lambda i, c: steady(i * NBUF, c), 0)
        for b in range(NBUF):
            drain(b)
            accumulate(b, bpw - NBUF + b)
        pltpu.sync_copy(out_v, out_hbm.at[pl.ds(base, bpw)])

    return sc_pool


def _head_body(s_ref, b1_ref, w2_ref, b2_ref, o_ref):
    h = jnp.maximum(s_ref[:] * (1.0 / HIST) + b1_ref[:], 0.0)
    o_ref[:] = jnp.dot(h, w2_ref[:], preferred_element_type=jnp.float32) + b2_ref[:]


def _head(sums, b1, W2, b2):
    hpk = PACK
    sums_pk = sums.reshape(BATCH // hpk, hpk * HID)
    b1big = jnp.tile(b1, (hpk,)).reshape(1, hpk * HID)
    w2big = jnp.kron(jnp.eye(hpk, dtype=W2.dtype), W2)
    b2big = jnp.tile(b2, (hpk,)).reshape(1, hpk * OUT)
    rows_blk = 256
    grid = BATCH // hpk // rows_blk
    out_pk = pl.pallas_call(
        _head_body,
        grid=(grid,),
        in_specs=[
            pl.BlockSpec((rows_blk, hpk * HID), lambda i: (i, 0)),
            pl.BlockSpec((1, hpk * HID), lambda i: (0, 0)),
            pl.BlockSpec((hpk * HID, hpk * OUT), lambda i: (0, 0)),
            pl.BlockSpec((1, hpk * OUT), lambda i: (0, 0)),
        ],
        out_specs=pl.BlockSpec((rows_blk, hpk * OUT), lambda i: (i, 0)),
        out_shape=jax.ShapeDtypeStruct((BATCH // hpk, hpk * OUT), jnp.float32),
    )(sums_pk, b1big, w2big, b2big)
    return out_pk.reshape(BATCH, OUT)


def kernel(x, table, W1, b1, W2, b2):
    table1 = _fold_table(table, W1)
    sums = _make_sc_pool()(x.astype(jnp.int32).reshape(-1), table1)
    return _head(sums, b1, W2, b2)
